# baseline (speedup 1.0000x reference)
import functools

import jax
import jax.numpy as jnp
from jax import lax
from jax.experimental import pallas as pl
from jax.experimental.pallas import tpu as pltpu
from jax.experimental.pallas import tpu_sc as plsc

N = 10000
E = 320000
D = 128
G = 16
NEG_BIG = -3.0e38

NTILES = 32
ROWS_PER = 320
NPAD = NTILES * ROWS_PER
BLK = 3200
NBLK = E // BLK
VEC_UNROLL = 8
CHUNK = 128
NBUF = 4
CAP = 3600


def _uv_body(x_ref, pos_ref, wxu_ref, wpu_ref, wxv_ref, wpv_ref, u_ref, v_ref):
    x = x_ref[...]
    p = pos_ref[...]
    u_ref[...] = (
        jnp.dot(x, wxu_ref[...], preferred_element_type=jnp.float32)
        + jnp.dot(p, wpu_ref[...], preferred_element_type=jnp.float32)
    )
    v_ref[...] = (
        jnp.dot(x, wxv_ref[...], preferred_element_type=jnp.float32)
        + jnp.dot(p, wpv_ref[...], preferred_element_type=jnp.float32)
    )


def _final_body(segmax_ref, u_ref, x_ref, batch_ref, bmsg_ref, gnw_ref, gnb_ref,
                gnm_ref, wfc_ref, bfc_ref, out_ref):
    segmax = segmax_ref[...]
    agg = jnp.where(segmax <= NEG_BIG,
                    0.0,
                    u_ref[...] + segmax + bmsg_ref[...])
    bt = batch_ref[...]
    gids = jax.lax.broadcasted_iota(jnp.int32, (G, 1), 0)
    onehot = (bt == gids).astype(jnp.float32)
    counts = jnp.maximum(jnp.sum(onehot, axis=1, keepdims=True), 1.0)
    mean = jnp.dot(onehot, agg, preferred_element_type=jnp.float32) / counts
    meanb = jnp.dot(onehot.T, mean * gnm_ref[...], preferred_element_type=jnp.float32)
    out = agg - meanb
    var = jnp.dot(onehot, out * out, preferred_element_type=jnp.float32) / counts
    stdb = jnp.sqrt(jnp.dot(onehot.T, var, preferred_element_type=jnp.float32) + 1e-5)
    out = gnw_ref[...] * out / stdb + gnb_ref[...]
    out = jnp.maximum(out, 0.0)
    out = jnp.dot(out, wfc_ref[...], preferred_element_type=jnp.float32)
    out_ref[...] = out + bfc_ref[...] + x_ref[...]


_GATHER_DNUMS = lax.GatherDimensionNumbers(
    offset_dims=(), collapsed_slice_dims=(0,), start_index_map=(0,))


def _lane_bcast(vec, k):
    idx = jnp.full((16,), k, dtype=jnp.int32)
    return lax.gather(vec, idx[:, None], _GATHER_DNUMS, slice_sizes=(1,),
                      mode=lax.GatherScatterMode.PROMISE_IN_BOUNDS)


def _sc_segmax(v, src, dst):
    mesh = plsc.VectorSubcoreMesh(core_axis_name="c", subcore_axis_name="s")

    @functools.partial(
        pl.kernel,
        out_type=jax.ShapeDtypeStruct((NPAD, D), jnp.float32),
        mesh=mesh,
        compiler_params=pltpu.CompilerParams(needs_layout_passes=False,
                                             use_tc_tiling_on_sc=False),
        scratch_types=[
            pltpu.VMEM((2, BLK), jnp.int32),
            pltpu.VMEM((2, BLK), jnp.int32),
            pltpu.VMEM((CAP,), jnp.int32),
            pltpu.VMEM((CAP,), jnp.int32),
            pltpu.VMEM((CHUNK,), jnp.int32),
            pltpu.VMEM((CHUNK,), jnp.int32),
            pltpu.VMEM((CHUNK,), jnp.int32),
            pltpu.VMEM((CHUNK,), jnp.int32),
            pltpu.VMEM((NBUF, CHUNK, D), jnp.float32),
        ] + [pltpu.VMEM((ROWS_PER + 1, 16), jnp.float32)
             for _ in range(8)] + [
            pltpu.SemaphoreType.DMA,
            pltpu.SemaphoreType.DMA,
            pltpu.SemaphoreType.DMA,
        ],
    )
    def k(v_hbm, src_hbm, dst_hbm, out_hbm,
          dstblk, srcblk, cdst, csrc, idx0, idx1, idx2, idx3, rows,
          acc0, acc1, acc2, acc3, acc4, acc5, acc6, acc7,
          sem_d, sem_s, sem_g):
        accs = (acc0, acc1, acc2, acc3, acc4, acc5, acc6, acc7)
        idxs = (idx0, idx1, idx2, idx3)
        cid = lax.axis_index("c")
        sid = lax.axis_index("s")
        wid = sid * 2 + cid
        iota16 = lax.broadcasted_iota(jnp.int32, (16,), 0)
        lo = wid * ROWS_PER
        hi = lo + ROWS_PER
        negv = jnp.full((16,), NEG_BIG, dtype=jnp.float32)

        def ini(i, _):
            for cch in range(8):
                accs[cch][i, pl.ds(0, 16)] = negv
            return 0
        lax.fori_loop(0, ROWS_PER + 1, ini, 0)

        def start_block_dma(bi, p):
            pltpu.async_copy(dst_hbm.at[pl.ds(bi * BLK, BLK)],
                             dstblk.at[p], sem_d)
            pltpu.async_copy(src_hbm.at[pl.ds(bi * BLK, BLK)],
                             srcblk.at[p], sem_s)

        def wait_block_dma():
            pltpu.make_async_copy(dst_hbm.at[pl.ds(0, BLK)],
                                  dstblk.at[0], sem_d).wait()
            pltpu.make_async_copy(src_hbm.at[pl.ds(0, BLK)],
                                  srcblk.at[0], sem_s).wait()

        def stage_and_fire(slot0, q):
            for b in range(NBUF):
                @pl.when(q == b)
                def _(b=b):
                    for g2 in range(CHUNK // 16):
                        idxs[b][pl.ds(g2 * 16, 16)] = \
                            csrc[pl.ds(slot0 + g2 * 16, 16)]
                    pltpu.async_copy(v_hbm.at[idxs[b]], rows.at[b], sem_g)

        def wait_gather():
            pltpu.make_async_copy(v_hbm.at[idx0],
                                  rows.at[0], sem_g).wait()

        def process(slot0, q):
            def grp(g, _):
                rel16 = cdst[pl.ds(slot0 + g * 16, 16)] - lo
                rs = [rel16[kk] for kk in range(16)]
                for wave in range(2):
                    for t in range(8):
                        for e in range(8):
                            cch = (e + t) & 7
                            kk = wave * 8 + e
                            r = rs[kk]
                            rowk = g * 16 + kk
                            av = accs[cch][r, pl.ds(0, 16)]
                            rv = rows[q, rowk, pl.ds(cch * 16, 16)]
                            accs[cch][r, pl.ds(0, 16)] = jnp.maximum(av, rv)
                return 0
            lax.fori_loop(0, CHUNK // 16, grp, 0)

        def drain_chunks(nch):
            for b in range(NBUF - 1):
                @pl.when(nch > b)
                def _(b=b):
                    stage_and_fire(b * CHUNK, b)

            def chloop(i, _):
                q = i & (NBUF - 1)
                wait_gather()

                @pl.when(i + NBUF - 1 < nch)
                def _():
                    stage_and_fire((i + NBUF - 1) * CHUNK,
                                   (i + NBUF - 1) & (NBUF - 1))
                process(i * CHUNK, q)
                return 0
            lax.fori_loop(0, nch, chloop, 0)

        def block(b, cntv):
            p = b & 1
            wait_block_dma()

            @pl.when(b + 1 < NBLK)
            def _():
                start_block_dma(b + 1, 1 - p)

            def filt(vi, cntv):
                ms, cums, dstvs, srcvs = [], [], [], []
                for u in range(VEC_UNROLL):
                    off = (vi * VEC_UNROLL + u) * 16
                    dstv = dstblk[p, pl.ds(off, 16)]
                    srcv = srcblk[p, pl.ds(off, 16)]
                    m = (dstv >= lo) & (dstv < hi)
                    cum = plsc.cumsum(m.astype(jnp.int32))
                    ms.append(m)
                    cums.append(cum)
                    dstvs.append(dstv)
                    srcvs.append(srcv)
                pref = cntv
                for u in range(VEC_UNROLL):
                    pos = pref + cums[u] - 1
                    plsc.store_scatter(cdst, [pos], dstvs[u], mask=ms[u])
                    plsc.store_scatter(csrc, [pos], srcvs[u], mask=ms[u])
                    pref = pref + _lane_bcast(cums[u], 15)
                return pref
            cntv = lax.fori_loop(0, BLK // 16 // VEC_UNROLL, filt, cntv)
            s = jnp.max(cntv)
            nch = s // CHUNK
            drain_chunks(nch)
            rem = s - nch * CHUNK
            base = nch * CHUNK

            def cp(i, _):
                t1 = cdst[pl.ds(base + i * 16, 16)]
                cdst[pl.ds(i * 16, 16)] = t1
                t2 = csrc[pl.ds(base + i * 16, 16)]
                csrc[pl.ds(i * 16, 16)] = t2
                return 0
            lax.fori_loop(0, (rem + 15) // 16, cp, 0)
            return lax.broadcast(rem, (16,))

        start_block_dma(0, 0)
        cntv = lax.fori_loop(0, NBLK, block, jnp.zeros((16,), jnp.int32))

        s = jnp.max(cntv)
        dummy_dst = lax.broadcast(lo + ROWS_PER, (16,))
        zeros16 = jnp.zeros((16,), jnp.int32)
        for i in range(CHUNK // 16):
            posf = s + i * 16 + iota16
            plsc.store_scatter(cdst, [posf], dummy_dst)
            plsc.store_scatter(csrc, [posf], zeros16)
        drain_chunks((s + CHUNK - 1) // CHUNK)

        for cch in range(8):
            pltpu.sync_copy(
                accs[cch].at[pl.ds(0, ROWS_PER)],
                out_hbm.at[pl.ds(lo, ROWS_PER), pl.ds(cch * 16, 16)])

    return k(v, src, dst)


def kernel(x, pos, edge_index, batch, W_msg, b_msg, gn_weight, gn_bias,
           gn_mean_scale, W_fc, b_fc):
    src = edge_index[0]
    dst = edge_index[1]
    W1 = W_msg[:D]
    W2 = W_msg[D:2 * D]
    W3 = W_msg[2 * D:]

    u, v = pl.pallas_call(
        _uv_body,
        out_shape=(jax.ShapeDtypeStruct((N, D), jnp.float32),
                   jax.ShapeDtypeStruct((N, D), jnp.float32)),
    )(x, pos, W1 - W2, -W3, W2, W3)

    segmax = _sc_segmax(v, src, dst)[:N]

    out = pl.pallas_call(
        _final_body,
        out_shape=jax.ShapeDtypeStruct((N, D), jnp.float32),
    )(segmax, u, x, batch.reshape(1, N), b_msg.reshape(1, D),
      gn_weight.reshape(1, D), gn_bias.reshape(1, D),
      gn_mean_scale.reshape(1, D), W_fc, b_fc.reshape(1, D))
    return out


# speedup vs baseline: 1.2256x; 1.2256x over previous
import functools

import jax
import jax.numpy as jnp
from jax import lax
from jax.experimental import pallas as pl
from jax.experimental.pallas import tpu as pltpu
from jax.experimental.pallas import tpu_sc as plsc

N = 10000
E = 320000
D = 128
G = 16
NEG_BIG = -3.0e38

NTILES = 32
ROWS_PER = 320
NPAD = NTILES * ROWS_PER
BLK = 3200
NBLK = E // BLK
VEC_UNROLL = 8
CHUNK = 128
NBUF = 4
CAP = 3600


def _uv_body(x_ref, pos_ref, wxu_ref, wpu_ref, wxv_ref, wpv_ref, u_ref, v_ref):
    x = x_ref[...]
    p = pos_ref[...]
    u_ref[...] = (
        jnp.dot(x, wxu_ref[...], preferred_element_type=jnp.float32)
        + jnp.dot(p, wpu_ref[...], preferred_element_type=jnp.float32)
    )
    v_ref[...] = (
        jnp.dot(x, wxv_ref[...], preferred_element_type=jnp.float32)
        + jnp.dot(p, wpv_ref[...], preferred_element_type=jnp.float32)
    )


def _final_body(segmax_ref, u_ref, x_ref, batch_ref, bmsg_ref, gnw_ref, gnb_ref,
                gnm_ref, wfc_ref, bfc_ref, out_ref):
    segmax = segmax_ref[...]
    agg = jnp.where(segmax <= NEG_BIG,
                    0.0,
                    u_ref[...] + segmax + bmsg_ref[...])
    bt = batch_ref[...]
    gids = jax.lax.broadcasted_iota(jnp.int32, (G, 1), 0)
    onehot = (bt == gids).astype(jnp.float32)
    counts = jnp.maximum(jnp.sum(onehot, axis=1, keepdims=True), 1.0)
    mean = jnp.dot(onehot, agg, preferred_element_type=jnp.float32) / counts
    meanb = jnp.dot(onehot.T, mean * gnm_ref[...], preferred_element_type=jnp.float32)
    out = agg - meanb
    var = jnp.dot(onehot, out * out, preferred_element_type=jnp.float32) / counts
    stdb = jnp.sqrt(jnp.dot(onehot.T, var, preferred_element_type=jnp.float32) + 1e-5)
    out = gnw_ref[...] * out / stdb + gnb_ref[...]
    out = jnp.maximum(out, 0.0)
    out = jnp.dot(out, wfc_ref[...], preferred_element_type=jnp.float32)
    out_ref[...] = out + bfc_ref[...] + x_ref[...]


_GATHER_DNUMS = lax.GatherDimensionNumbers(
    offset_dims=(), collapsed_slice_dims=(0,), start_index_map=(0,))


def _lane_bcast(vec, k):
    idx = jnp.full((16,), k, dtype=jnp.int32)
    return lax.gather(vec, idx[:, None], _GATHER_DNUMS, slice_sizes=(1,),
                      mode=lax.GatherScatterMode.PROMISE_IN_BOUNDS)


def _sc_segmax(v, src, dst):
    mesh = plsc.VectorSubcoreMesh(core_axis_name="c", subcore_axis_name="s")

    @functools.partial(
        pl.kernel,
        out_type=jax.ShapeDtypeStruct((NPAD, D), jnp.float32),
        mesh=mesh,
        compiler_params=pltpu.CompilerParams(needs_layout_passes=False,
                                             use_tc_tiling_on_sc=False),
        scratch_types=[
            pltpu.VMEM((2, BLK), jnp.int32),
            pltpu.VMEM((2, BLK), jnp.int32),
            pltpu.VMEM((CAP,), jnp.int32),
            pltpu.VMEM((CAP,), jnp.int32),
            pltpu.VMEM((CHUNK,), jnp.int32),
            pltpu.VMEM((CHUNK,), jnp.int32),
            pltpu.VMEM((CHUNK,), jnp.int32),
            pltpu.VMEM((CHUNK,), jnp.int32),
            pltpu.VMEM((NBUF, CHUNK, D), jnp.float32),
            pltpu.VMEM((ROWS_PER + 1, D), jnp.float32),
            pltpu.SMEM((CHUNK,), jnp.int32),
        ] + [
            pltpu.SemaphoreType.DMA,
            pltpu.SemaphoreType.DMA,
            pltpu.SemaphoreType.DMA,
        ],
    )
    def k(v_hbm, src_hbm, dst_hbm, out_hbm,
          dstblk, srcblk, cdst, csrc, idx0, idx1, idx2, idx3, rows,
          acc, rel_smem, sem_d, sem_s, sem_g):
        idxs = (idx0, idx1, idx2, idx3)
        cid = lax.axis_index("c")
        sid = lax.axis_index("s")
        wid = sid * 2 + cid
        iota16 = lax.broadcasted_iota(jnp.int32, (16,), 0)
        lo = wid * ROWS_PER
        hi = lo + ROWS_PER
        negv = jnp.full((16,), NEG_BIG, dtype=jnp.float32)

        def ini(i, _):
            for cch in range(8):
                acc[i, pl.ds(cch * 16, 16)] = negv
            return 0
        lax.fori_loop(0, ROWS_PER + 1, ini, 0)

        def start_block_dma(bi, p):
            pltpu.async_copy(dst_hbm.at[pl.ds(bi * BLK, BLK)],
                             dstblk.at[p], sem_d)
            pltpu.async_copy(src_hbm.at[pl.ds(bi * BLK, BLK)],
                             srcblk.at[p], sem_s)

        def wait_block_dma():
            pltpu.make_async_copy(dst_hbm.at[pl.ds(0, BLK)],
                                  dstblk.at[0], sem_d).wait()
            pltpu.make_async_copy(src_hbm.at[pl.ds(0, BLK)],
                                  srcblk.at[0], sem_s).wait()

        def stage_and_fire(slot0, q):
            for b in range(NBUF):
                @pl.when(q == b)
                def _(b=b):
                    for g2 in range(CHUNK // 16):
                        idxs[b][pl.ds(g2 * 16, 16)] = \
                            csrc[pl.ds(slot0 + g2 * 16, 16)]
                    pltpu.async_copy(v_hbm.at[idxs[b]], rows.at[b], sem_g)

        def wait_gather():
            pltpu.make_async_copy(v_hbm.at[idx0],
                                  rows.at[0], sem_g).wait()

        def process(slot0, q):
            for g in range(CHUNK // 16):
                rel16 = cdst[pl.ds(slot0 + g * 16, 16)] - lo
                for kk in range(16):
                    rel_smem[g * 16 + kk] = rel16[kk]

            @plsc.parallel_loop(0, 8, 1, unroll=2)
            def _(cch):
                col = cch * 16

                def egrp(g, _):
                    for kk in range(16):
                        r = rel_smem[g * 16 + kk]
                        av = acc[r, pl.ds(col, 16)]
                        rv = rows[q, g * 16 + kk, pl.ds(col, 16)]
                        acc[r, pl.ds(col, 16)] = jnp.maximum(av, rv)
                    return 0
                lax.fori_loop(0, CHUNK // 16, egrp, 0)

        def drain_chunks(nch):
            for b in range(NBUF - 1):
                @pl.when(nch > b)
                def _(b=b):
                    stage_and_fire(b * CHUNK, b)

            def chloop(i, _):
                q = i & (NBUF - 1)
                wait_gather()

                @pl.when(i + NBUF - 1 < nch)
                def _():
                    stage_and_fire((i + NBUF - 1) * CHUNK,
                                   (i + NBUF - 1) & (NBUF - 1))
                process(i * CHUNK, q)
                return 0
            lax.fori_loop(0, nch, chloop, 0)

        def block(b, cntv):
            p = b & 1
            wait_block_dma()

            @pl.when(b + 1 < NBLK)
            def _():
                start_block_dma(b + 1, 1 - p)

            def filt(vi, cntv):
                ms, cums, dstvs, srcvs = [], [], [], []
                for u in range(VEC_UNROLL):
                    off = (vi * VEC_UNROLL + u) * 16
                    dstv = dstblk[p, pl.ds(off, 16)]
                    srcv = srcblk[p, pl.ds(off, 16)]
                    m = (dstv >= lo) & (dstv < hi)
                    cum = plsc.cumsum(m.astype(jnp.int32))
                    ms.append(m)
                    cums.append(cum)
                    dstvs.append(dstv)
                    srcvs.append(srcv)
                pref = cntv
                for u in range(VEC_UNROLL):
                    pos = pref + cums[u] - 1
                    plsc.store_scatter(cdst, [pos], dstvs[u], mask=ms[u])
                    plsc.store_scatter(csrc, [pos], srcvs[u], mask=ms[u])
                    pref = pref + _lane_bcast(cums[u], 15)
                return pref
            cntv = lax.fori_loop(0, BLK // 16 // VEC_UNROLL, filt, cntv)
            s = jnp.max(cntv)
            nch = s // CHUNK
            drain_chunks(nch)
            rem = s - nch * CHUNK
            base = nch * CHUNK

            def cp(i, _):
                t1 = cdst[pl.ds(base + i * 16, 16)]
                cdst[pl.ds(i * 16, 16)] = t1
                t2 = csrc[pl.ds(base + i * 16, 16)]
                csrc[pl.ds(i * 16, 16)] = t2
                return 0
            lax.fori_loop(0, (rem + 15) // 16, cp, 0)
            return lax.broadcast(rem, (16,))

        start_block_dma(0, 0)
        cntv = lax.fori_loop(0, NBLK, block, jnp.zeros((16,), jnp.int32))

        s = jnp.max(cntv)
        dummy_dst = lax.broadcast(lo + ROWS_PER, (16,))
        zeros16 = jnp.zeros((16,), jnp.int32)
        for i in range(CHUNK // 16):
            posf = s + i * 16 + iota16
            plsc.store_scatter(cdst, [posf], dummy_dst)
            plsc.store_scatter(csrc, [posf], zeros16)
        drain_chunks((s + CHUNK - 1) // CHUNK)

        pltpu.sync_copy(acc.at[pl.ds(0, ROWS_PER)],
                        out_hbm.at[pl.ds(lo, ROWS_PER)])

    return k(v, src, dst)


def kernel(x, pos, edge_index, batch, W_msg, b_msg, gn_weight, gn_bias,
           gn_mean_scale, W_fc, b_fc):
    src = edge_index[0]
    dst = edge_index[1]
    W1 = W_msg[:D]
    W2 = W_msg[D:2 * D]
    W3 = W_msg[2 * D:]

    u, v = pl.pallas_call(
        _uv_body,
        out_shape=(jax.ShapeDtypeStruct((N, D), jnp.float32),
                   jax.ShapeDtypeStruct((N, D), jnp.float32)),
    )(x, pos, W1 - W2, -W3, W2, W3)

    segmax = _sc_segmax(v, src, dst)[:N]

    out = pl.pallas_call(
        _final_body,
        out_shape=jax.ShapeDtypeStruct((N, D), jnp.float32),
    )(segmax, u, x, batch.reshape(1, N), b_msg.reshape(1, D),
      gn_weight.reshape(1, D), gn_bias.reshape(1, D),
      gn_mean_scale.reshape(1, D), W_fc, b_fc.reshape(1, D))
    return out


# speedup vs baseline: 1.5675x; 1.2790x over previous
import functools

import jax
import jax.numpy as jnp
from jax import lax
from jax.experimental import pallas as pl
from jax.experimental.pallas import tpu as pltpu
from jax.experimental.pallas import tpu_sc as plsc

N = 10000
E = 320000
D = 128
G = 16
NEG_BIG = -3.0e38

NTILES = 32
ROWS_PER = 320
NPAD = NTILES * ROWS_PER
BLK = 3200
NBLK = E // BLK
VEC_UNROLL = 8
CHUNK = 128
NBUF = 4
CAP = 3600


def _uv_body(x_ref, pos_ref, wxu_ref, wpu_ref, wxv_ref, wpv_ref, u_ref, v_ref):
    x = x_ref[...]
    p = pos_ref[...]
    u_ref[...] = (
        jnp.dot(x, wxu_ref[...], preferred_element_type=jnp.float32)
        + jnp.dot(p, wpu_ref[...], preferred_element_type=jnp.float32)
    )
    v_ref[...] = (
        jnp.dot(x, wxv_ref[...], preferred_element_type=jnp.float32)
        + jnp.dot(p, wpv_ref[...], preferred_element_type=jnp.float32)
    )


def _final_body(segmax_ref, u_ref, x_ref, batch_ref, bmsg_ref, gnw_ref, gnb_ref,
                gnm_ref, wfc_ref, bfc_ref, out_ref):
    segmax = segmax_ref[...]
    agg = jnp.where(segmax <= NEG_BIG,
                    0.0,
                    u_ref[...] + segmax + bmsg_ref[...])
    bt = batch_ref[...]
    gids = jax.lax.broadcasted_iota(jnp.int32, (G, 1), 0)
    onehot = (bt == gids).astype(jnp.float32)
    counts = jnp.maximum(jnp.sum(onehot, axis=1, keepdims=True), 1.0)
    mean = jnp.dot(onehot, agg, preferred_element_type=jnp.float32) / counts
    meanb = jnp.dot(onehot.T, mean * gnm_ref[...], preferred_element_type=jnp.float32)
    out = agg - meanb
    var = jnp.dot(onehot, out * out, preferred_element_type=jnp.float32) / counts
    stdb = jnp.sqrt(jnp.dot(onehot.T, var, preferred_element_type=jnp.float32) + 1e-5)
    out = gnw_ref[...] * out / stdb + gnb_ref[...]
    out = jnp.maximum(out, 0.0)
    out = jnp.dot(out, wfc_ref[...], preferred_element_type=jnp.float32)
    out_ref[...] = out + bfc_ref[...] + x_ref[...]


_GATHER_DNUMS = lax.GatherDimensionNumbers(
    offset_dims=(), collapsed_slice_dims=(0,), start_index_map=(0,))


def _lane_bcast(vec, k):
    idx = jnp.full((16,), k, dtype=jnp.int32)
    return lax.gather(vec, idx[:, None], _GATHER_DNUMS, slice_sizes=(1,),
                      mode=lax.GatherScatterMode.PROMISE_IN_BOUNDS)


def _sc_segmax(v, src, dst):
    mesh = plsc.VectorSubcoreMesh(core_axis_name="c", subcore_axis_name="s")

    @functools.partial(
        pl.kernel,
        out_type=jax.ShapeDtypeStruct((NPAD, D), jnp.float32),
        mesh=mesh,
        compiler_params=pltpu.CompilerParams(needs_layout_passes=False,
                                             use_tc_tiling_on_sc=False),
        scratch_types=[
            pltpu.VMEM((2, BLK), jnp.int32),
            pltpu.VMEM((2, BLK), jnp.int32),
            pltpu.VMEM((CAP,), jnp.int32),
            pltpu.VMEM((CAP,), jnp.int32),
            pltpu.VMEM((CHUNK,), jnp.int32),
            pltpu.VMEM((CHUNK,), jnp.int32),
            pltpu.VMEM((CHUNK,), jnp.int32),
            pltpu.VMEM((CHUNK,), jnp.int32),
            pltpu.VMEM((NBUF, CHUNK, D), jnp.float32),
            pltpu.VMEM((ROWS_PER + 1, D), jnp.float32),
            pltpu.SMEM((CHUNK,), jnp.int32),
        ] + [
            pltpu.SemaphoreType.DMA,
            pltpu.SemaphoreType.DMA,
            pltpu.SemaphoreType.DMA,
        ],
    )
    def k(v_hbm, src_hbm, dst_hbm, out_hbm,
          dstblk, srcblk, cdst, csrc, idx0, idx1, idx2, idx3, rows,
          acc, rel_smem, sem_d, sem_s, sem_g):
        idxs = (idx0, idx1, idx2, idx3)
        cid = lax.axis_index("c")
        sid = lax.axis_index("s")
        wid = sid * 2 + cid
        iota16 = lax.broadcasted_iota(jnp.int32, (16,), 0)
        lo = wid * ROWS_PER
        hi = lo + ROWS_PER
        negv = jnp.full((16,), NEG_BIG, dtype=jnp.float32)

        def ini(i, _):
            for cch in range(8):
                acc[i, pl.ds(cch * 16, 16)] = negv
            return 0
        lax.fori_loop(0, ROWS_PER + 1, ini, 0)

        def start_block_dma(bi, p):
            pltpu.async_copy(dst_hbm.at[pl.ds(bi * BLK, BLK)],
                             dstblk.at[p], sem_d)
            pltpu.async_copy(src_hbm.at[pl.ds(bi * BLK, BLK)],
                             srcblk.at[p], sem_s)

        def wait_block_dma():
            pltpu.make_async_copy(dst_hbm.at[pl.ds(0, BLK)],
                                  dstblk.at[0], sem_d).wait()
            pltpu.make_async_copy(src_hbm.at[pl.ds(0, BLK)],
                                  srcblk.at[0], sem_s).wait()

        def stage_and_fire(slot0, q):
            for b in range(NBUF):
                @pl.when(q == b)
                def _(b=b):
                    for g2 in range(CHUNK // 16):
                        idxs[b][pl.ds(g2 * 16, 16)] = \
                            csrc[pl.ds(slot0 + g2 * 16, 16)]
                    pltpu.async_copy(v_hbm.at[idxs[b]], rows.at[b], sem_g)

        def wait_gather():
            pltpu.make_async_copy(v_hbm.at[idx0],
                                  rows.at[0], sem_g).wait()

        def process(slot0, q):
            def grp(g, _):
                rel16 = cdst[pl.ds(slot0 + g * 16, 16)] - lo
                for kk in range(16):
                    r = rel16[kk]
                    rowk = g * 16 + kk
                    avs = [acc[r, pl.ds(c * 16, 16)] for c in range(8)]
                    rvs = [rows[q, rowk, pl.ds(c * 16, 16)] for c in range(8)]
                    for c in range(8):
                        acc[r, pl.ds(c * 16, 16)] = jnp.maximum(avs[c], rvs[c])
                return 0
            lax.fori_loop(0, CHUNK // 16, grp, 0)

        def drain_chunks(nch):
            for b in range(NBUF - 1):
                @pl.when(nch > b)
                def _(b=b):
                    stage_and_fire(b * CHUNK, b)

            def chloop(i, _):
                q = i & (NBUF - 1)
                wait_gather()

                @pl.when(i + NBUF - 1 < nch)
                def _():
                    stage_and_fire((i + NBUF - 1) * CHUNK,
                                   (i + NBUF - 1) & (NBUF - 1))
                process(i * CHUNK, q)
                return 0
            lax.fori_loop(0, nch, chloop, 0)

        def block(b, cntv):
            p = b & 1
            wait_block_dma()

            @pl.when(b + 1 < NBLK)
            def _():
                start_block_dma(b + 1, 1 - p)

            def filt(vi, cntv):
                ms, cums, dstvs, srcvs = [], [], [], []
                for u in range(VEC_UNROLL):
                    off = (vi * VEC_UNROLL + u) * 16
                    dstv = dstblk[p, pl.ds(off, 16)]
                    srcv = srcblk[p, pl.ds(off, 16)]
                    m = (dstv >= lo) & (dstv < hi)
                    cum = plsc.cumsum(m.astype(jnp.int32))
                    ms.append(m)
                    cums.append(cum)
                    dstvs.append(dstv)
                    srcvs.append(srcv)
                pref = cntv
                for u in range(VEC_UNROLL):
                    pos = pref + cums[u] - 1
                    plsc.store_scatter(cdst, [pos], dstvs[u], mask=ms[u])
                    plsc.store_scatter(csrc, [pos], srcvs[u], mask=ms[u])
                    pref = pref + _lane_bcast(cums[u], 15)
                return pref
            cntv = lax.fori_loop(0, BLK // 16 // VEC_UNROLL, filt, cntv)
            s = jnp.max(cntv)
            nch = s // CHUNK
            drain_chunks(nch)
            rem = s - nch * CHUNK
            base = nch * CHUNK

            def cp(i, _):
                t1 = cdst[pl.ds(base + i * 16, 16)]
                cdst[pl.ds(i * 16, 16)] = t1
                t2 = csrc[pl.ds(base + i * 16, 16)]
                csrc[pl.ds(i * 16, 16)] = t2
                return 0
            lax.fori_loop(0, (rem + 15) // 16, cp, 0)
            return lax.broadcast(rem, (16,))

        start_block_dma(0, 0)
        cntv = lax.fori_loop(0, NBLK, block, jnp.zeros((16,), jnp.int32))

        s = jnp.max(cntv)
        dummy_dst = lax.broadcast(lo + ROWS_PER, (16,))
        zeros16 = jnp.zeros((16,), jnp.int32)
        for i in range(CHUNK // 16):
            posf = s + i * 16 + iota16
            plsc.store_scatter(cdst, [posf], dummy_dst)
            plsc.store_scatter(csrc, [posf], zeros16)
        drain_chunks((s + CHUNK - 1) // CHUNK)

        pltpu.sync_copy(acc.at[pl.ds(0, ROWS_PER)],
                        out_hbm.at[pl.ds(lo, ROWS_PER)])

    return k(v, src, dst)


def kernel(x, pos, edge_index, batch, W_msg, b_msg, gn_weight, gn_bias,
           gn_mean_scale, W_fc, b_fc):
    src = edge_index[0]
    dst = edge_index[1]
    W1 = W_msg[:D]
    W2 = W_msg[D:2 * D]
    W3 = W_msg[2 * D:]

    u, v = pl.pallas_call(
        _uv_body,
        out_shape=(jax.ShapeDtypeStruct((N, D), jnp.float32),
                   jax.ShapeDtypeStruct((N, D), jnp.float32)),
    )(x, pos, W1 - W2, -W3, W2, W3)

    segmax = _sc_segmax(v, src, dst)[:N]

    out = pl.pallas_call(
        _final_body,
        out_shape=jax.ShapeDtypeStruct((N, D), jnp.float32),
    )(segmax, u, x, batch.reshape(1, N), b_msg.reshape(1, D),
      gn_weight.reshape(1, D), gn_bias.reshape(1, D),
      gn_mean_scale.reshape(1, D), W_fc, b_fc.reshape(1, D))
    return out


# speedup vs baseline: 1.9354x; 1.2347x over previous
import functools

import jax
import jax.numpy as jnp
from jax import lax
from jax.experimental import pallas as pl
from jax.experimental.pallas import tpu as pltpu
from jax.experimental.pallas import tpu_sc as plsc

N = 10000
E = 320000
D = 128
G = 16
NEG_BIG = -3.0e38

NTILES = 32
ROWS_PER = 320
NPAD = NTILES * ROWS_PER
BLK = 3200
NBLK = E // BLK
VEC_UNROLL = 8
CHUNK = 128
NBUF = 4
CAP = 3600


def _uv_body(x_ref, pos_ref, wxu_ref, wpu_ref, wxv_ref, wpv_ref, u_ref, v_ref):
    x = x_ref[...]
    p = pos_ref[...]
    u_ref[...] = (
        jnp.dot(x, wxu_ref[...], preferred_element_type=jnp.float32)
        + jnp.dot(p, wpu_ref[...], preferred_element_type=jnp.float32)
    )
    v_ref[...] = (
        jnp.dot(x, wxv_ref[...], preferred_element_type=jnp.float32)
        + jnp.dot(p, wpv_ref[...], preferred_element_type=jnp.float32)
    )


def _final_body(segmax_ref, u_ref, x_ref, batch_ref, bmsg_ref, gnw_ref, gnb_ref,
                gnm_ref, wfc_ref, bfc_ref, out_ref):
    segmax = segmax_ref[...]
    agg = jnp.where(segmax <= NEG_BIG,
                    0.0,
                    u_ref[...] + segmax + bmsg_ref[...])
    bt = batch_ref[...]
    gids = jax.lax.broadcasted_iota(jnp.int32, (G, 1), 0)
    onehot = (bt == gids).astype(jnp.float32)
    counts = jnp.maximum(jnp.sum(onehot, axis=1, keepdims=True), 1.0)
    mean = jnp.dot(onehot, agg, preferred_element_type=jnp.float32) / counts
    meanb = jnp.dot(onehot.T, mean * gnm_ref[...], preferred_element_type=jnp.float32)
    out = agg - meanb
    var = jnp.dot(onehot, out * out, preferred_element_type=jnp.float32) / counts
    stdb = jnp.sqrt(jnp.dot(onehot.T, var, preferred_element_type=jnp.float32) + 1e-5)
    out = gnw_ref[...] * out / stdb + gnb_ref[...]
    out = jnp.maximum(out, 0.0)
    out = jnp.dot(out, wfc_ref[...], preferred_element_type=jnp.float32)
    out_ref[...] = out + bfc_ref[...] + x_ref[...]


_GATHER_DNUMS = lax.GatherDimensionNumbers(
    offset_dims=(), collapsed_slice_dims=(0,), start_index_map=(0,))


def _lane_bcast(vec, k):
    idx = jnp.full((16,), k, dtype=jnp.int32)
    return lax.gather(vec, idx[:, None], _GATHER_DNUMS, slice_sizes=(1,),
                      mode=lax.GatherScatterMode.PROMISE_IN_BOUNDS)


def _sc_segmax(v, src, dst):
    mesh = plsc.VectorSubcoreMesh(core_axis_name="c", subcore_axis_name="s")

    @functools.partial(
        pl.kernel,
        out_type=jax.ShapeDtypeStruct((NPAD, D), jnp.float32),
        mesh=mesh,
        compiler_params=pltpu.CompilerParams(needs_layout_passes=False,
                                             use_tc_tiling_on_sc=False),
        scratch_types=[
            pltpu.VMEM((2, BLK), jnp.int32),
            pltpu.VMEM((2, BLK), jnp.int32),
            pltpu.VMEM((CAP,), jnp.int32),
            pltpu.VMEM((CAP,), jnp.int32),
            pltpu.VMEM((CHUNK,), jnp.int32),
            pltpu.VMEM((CHUNK,), jnp.int32),
            pltpu.VMEM((CHUNK,), jnp.int32),
            pltpu.VMEM((CHUNK,), jnp.int32),
            pltpu.VMEM((NBUF, CHUNK, D), jnp.float32),
            pltpu.VMEM((ROWS_PER + 1, D), jnp.float32),
            pltpu.VMEM((NBUF, CHUNK), jnp.int32),
        ] + [
            pltpu.SemaphoreType.DMA,
            pltpu.SemaphoreType.DMA,
            pltpu.SemaphoreType.DMA,
        ],
    )
    def k(v_hbm, src_hbm, dst_hbm, out_hbm,
          dstblk, srcblk, cdst, csrc, idx0, idx1, idx2, idx3, rows,
          acc, dstst, sem_d, sem_s, sem_g):
        idxs = (idx0, idx1, idx2, idx3)
        cid = lax.axis_index("c")
        sid = lax.axis_index("s")
        wid = sid * 2 + cid
        iota16 = lax.broadcasted_iota(jnp.int32, (16,), 0)
        lo = wid * ROWS_PER
        hi = lo + ROWS_PER
        negv = jnp.full((16,), NEG_BIG, dtype=jnp.float32)

        def ini(i, _):
            for cch in range(8):
                acc[i, pl.ds(cch * 16, 16)] = negv
            return 0
        lax.fori_loop(0, ROWS_PER + 1, ini, 0)

        def start_block_dma(bi, p):
            pltpu.async_copy(dst_hbm.at[pl.ds(bi * BLK, BLK)],
                             dstblk.at[p], sem_d)
            pltpu.async_copy(src_hbm.at[pl.ds(bi * BLK, BLK)],
                             srcblk.at[p], sem_s)

        def wait_block_dma():
            pltpu.make_async_copy(dst_hbm.at[pl.ds(0, BLK)],
                                  dstblk.at[0], sem_d).wait()
            pltpu.make_async_copy(src_hbm.at[pl.ds(0, BLK)],
                                  srcblk.at[0], sem_s).wait()

        def stage_and_fire(slot0, q):
            for b in range(NBUF):
                @pl.when(q == b)
                def _(b=b):
                    for g2 in range(CHUNK // 16):
                        idxs[b][pl.ds(g2 * 16, 16)] = \
                            csrc[pl.ds(slot0 + g2 * 16, 16)]
                        dstst[b, pl.ds(g2 * 16, 16)] = \
                            cdst[pl.ds(slot0 + g2 * 16, 16)]
                    pltpu.async_copy(v_hbm.at[idxs[b]], rows.at[b], sem_g)

        def wait_gather():
            pltpu.make_async_copy(v_hbm.at[idx0],
                                  rows.at[0], sem_g).wait()

        def process(q):
            def grp(g, _):
                rel16 = dstst[q, pl.ds(g * 16, 16)] - lo
                for kk in range(16):
                    r = rel16[kk]
                    rowk = g * 16 + kk
                    avs = [acc[r, pl.ds(c * 16, 16)] for c in range(8)]
                    rvs = [rows[q, rowk, pl.ds(c * 16, 16)] for c in range(8)]
                    for c in range(8):
                        acc[r, pl.ds(c * 16, 16)] = jnp.maximum(avs[c], rvs[c])
                return 0
            lax.fori_loop(0, CHUNK // 16, grp, 0)

        def fire_loop(nch, state):
            def body(ch, st):
                qf, qp, infl = st
                do_proc = infl >= NBUF - 1

                @pl.when(do_proc)
                def _():
                    wait_gather()
                    process(qp & (NBUF - 1))
                stage_and_fire(ch * CHUNK, qf & (NBUF - 1))
                d = do_proc.astype(jnp.int32)
                return (qf + 1, qp + d, infl + 1 - d)
            return lax.fori_loop(0, nch, body, state)

        def block(b, carry):
            cntv, qf, qp, infl = carry
            p = b & 1
            wait_block_dma()

            @pl.when(b + 1 < NBLK)
            def _():
                start_block_dma(b + 1, 1 - p)

            def filt(vi, cntv):
                ms, cums, dstvs, srcvs = [], [], [], []
                for u in range(VEC_UNROLL):
                    off = (vi * VEC_UNROLL + u) * 16
                    dstv = dstblk[p, pl.ds(off, 16)]
                    srcv = srcblk[p, pl.ds(off, 16)]
                    m = (dstv >= lo) & (dstv < hi)
                    cum = plsc.cumsum(m.astype(jnp.int32))
                    ms.append(m)
                    cums.append(cum)
                    dstvs.append(dstv)
                    srcvs.append(srcv)
                pref = cntv
                for u in range(VEC_UNROLL):
                    pos = pref + cums[u] - 1
                    plsc.store_scatter(cdst, [pos], dstvs[u], mask=ms[u])
                    plsc.store_scatter(csrc, [pos], srcvs[u], mask=ms[u])
                    pref = pref + _lane_bcast(cums[u], 15)
                return pref
            cntv = lax.fori_loop(0, BLK // 16 // VEC_UNROLL, filt, cntv)
            s = jnp.max(cntv)
            nch = s // CHUNK
            qf, qp, infl = fire_loop(nch, (qf, qp, infl))
            rem = s - nch * CHUNK
            base = nch * CHUNK

            def cp(i, _):
                t1 = cdst[pl.ds(base + i * 16, 16)]
                cdst[pl.ds(i * 16, 16)] = t1
                t2 = csrc[pl.ds(base + i * 16, 16)]
                csrc[pl.ds(i * 16, 16)] = t2
                return 0
            lax.fori_loop(0, (rem + 15) // 16, cp, 0)
            return (lax.broadcast(rem, (16,)), qf, qp, infl)

        start_block_dma(0, 0)
        zero = jnp.int32(0)
        cntv, qf, qp, infl = lax.fori_loop(
            0, NBLK, block, (jnp.zeros((16,), jnp.int32), zero, zero, zero))

        s = jnp.max(cntv)
        dummy_dst = lax.broadcast(lo + ROWS_PER, (16,))
        zeros16 = jnp.zeros((16,), jnp.int32)
        for i in range(CHUNK // 16):
            posf = s + i * 16 + iota16
            plsc.store_scatter(cdst, [posf], dummy_dst)
            plsc.store_scatter(csrc, [posf], zeros16)
        qf, qp, infl = fire_loop((s + CHUNK - 1) // CHUNK, (qf, qp, infl))

        def dr(i, qp2):
            wait_gather()
            process(qp2 & (NBUF - 1))
            return qp2 + 1
        lax.fori_loop(0, infl, dr, qp)

        pltpu.sync_copy(acc.at[pl.ds(0, ROWS_PER)],
                        out_hbm.at[pl.ds(lo, ROWS_PER)])

    return k(v, src, dst)


def kernel(x, pos, edge_index, batch, W_msg, b_msg, gn_weight, gn_bias,
           gn_mean_scale, W_fc, b_fc):
    src = edge_index[0]
    dst = edge_index[1]
    W1 = W_msg[:D]
    W2 = W_msg[D:2 * D]
    W3 = W_msg[2 * D:]

    u, v = pl.pallas_call(
        _uv_body,
        out_shape=(jax.ShapeDtypeStruct((N, D), jnp.float32),
                   jax.ShapeDtypeStruct((N, D), jnp.float32)),
    )(x, pos, W1 - W2, -W3, W2, W3)

    segmax = _sc_segmax(v, src, dst)[:N]

    out = pl.pallas_call(
        _final_body,
        out_shape=jax.ShapeDtypeStruct((N, D), jnp.float32),
    )(segmax, u, x, batch.reshape(1, N), b_msg.reshape(1, D),
      gn_weight.reshape(1, D), gn_bias.reshape(1, D),
      gn_mean_scale.reshape(1, D), W_fc, b_fc.reshape(1, D))
    return out


# speedup vs baseline: 1.9703x; 1.0180x over previous
import functools

import jax
import jax.numpy as jnp
from jax import lax
from jax.experimental import pallas as pl
from jax.experimental.pallas import tpu as pltpu
from jax.experimental.pallas import tpu_sc as plsc

N = 10000
E = 320000
D = 128
G = 16
NEG_BIG = -3.0e38

NTILES = 32
ROWS_PER = 320
NPAD = NTILES * ROWS_PER
BLK = 3200
NBLK = E // BLK
VEC_UNROLL = 8
CHUNK = 128
NBUF = 4
CAP = 3600


def _uv_body(x_ref, pos_ref, wxu_ref, wpu_ref, wxv_ref, wpv_ref, u_ref, v_ref):
    x = x_ref[...]
    p = pos_ref[...]
    u_ref[...] = (
        jnp.dot(x, wxu_ref[...], preferred_element_type=jnp.float32)
        + jnp.dot(p, wpu_ref[...], preferred_element_type=jnp.float32)
    )
    v_ref[...] = (
        jnp.dot(x, wxv_ref[...], preferred_element_type=jnp.float32)
        + jnp.dot(p, wpv_ref[...], preferred_element_type=jnp.float32)
    )


def _final_body(segmax_ref, u_ref, x_ref, batch_ref, bmsg_ref, gnw_ref, gnb_ref,
                gnm_ref, wfc_ref, bfc_ref, out_ref):
    segmax = segmax_ref[...]
    agg = jnp.where(segmax <= NEG_BIG,
                    0.0,
                    u_ref[...] + segmax + bmsg_ref[...])
    bt = batch_ref[...]
    gids = jax.lax.broadcasted_iota(jnp.int32, (G, 1), 0)
    onehot = (bt == gids).astype(jnp.float32)
    counts = jnp.maximum(jnp.sum(onehot, axis=1, keepdims=True), 1.0)
    mean = jnp.dot(onehot, agg, preferred_element_type=jnp.float32) / counts
    meanb = jnp.dot(onehot.T, mean * gnm_ref[...], preferred_element_type=jnp.float32)
    out = agg - meanb
    var = jnp.dot(onehot, out * out, preferred_element_type=jnp.float32) / counts
    stdb = jnp.sqrt(jnp.dot(onehot.T, var, preferred_element_type=jnp.float32) + 1e-5)
    out = gnw_ref[...] * out / stdb + gnb_ref[...]
    out = jnp.maximum(out, 0.0)
    out = jnp.dot(out, wfc_ref[...], preferred_element_type=jnp.float32)
    out_ref[...] = out + bfc_ref[...] + x_ref[...]


_GATHER_DNUMS = lax.GatherDimensionNumbers(
    offset_dims=(), collapsed_slice_dims=(0,), start_index_map=(0,))


def _lane_bcast(vec, k):
    idx = jnp.full((16,), k, dtype=jnp.int32)
    return lax.gather(vec, idx[:, None], _GATHER_DNUMS, slice_sizes=(1,),
                      mode=lax.GatherScatterMode.PROMISE_IN_BOUNDS)


def _sc_segmax(v, src, dst):
    mesh = plsc.VectorSubcoreMesh(core_axis_name="c", subcore_axis_name="s")

    @functools.partial(
        pl.kernel,
        out_type=jax.ShapeDtypeStruct((NPAD, D), jnp.float32),
        mesh=mesh,
        compiler_params=pltpu.CompilerParams(needs_layout_passes=False,
                                             use_tc_tiling_on_sc=False),
        scratch_types=[
            pltpu.VMEM((2, BLK), jnp.int32),
            pltpu.VMEM((2, BLK), jnp.int32),
            pltpu.VMEM((CAP,), jnp.int32),
            pltpu.VMEM((CHUNK,), jnp.int32),
            pltpu.VMEM((CHUNK,), jnp.int32),
            pltpu.VMEM((CHUNK,), jnp.int32),
            pltpu.VMEM((CHUNK,), jnp.int32),
            pltpu.VMEM((NBUF, CHUNK, D), jnp.float32),
            pltpu.VMEM((ROWS_PER + 1, D), jnp.float32),
            pltpu.VMEM((NBUF, CHUNK), jnp.int32),
        ] + [
            pltpu.SemaphoreType.DMA,
            pltpu.SemaphoreType.DMA,
            pltpu.SemaphoreType.DMA,
        ],
    )
    def k(v_hbm, src_hbm, dst_hbm, out_hbm,
          dstblk, srcblk, cpk, idx0, idx1, idx2, idx3, rows,
          acc, dstst, sem_d, sem_s, sem_g):
        idxs = (idx0, idx1, idx2, idx3)
        cid = lax.axis_index("c")
        sid = lax.axis_index("s")
        wid = sid * 2 + cid
        iota16 = lax.broadcasted_iota(jnp.int32, (16,), 0)
        lo = wid * ROWS_PER
        hi = lo + ROWS_PER
        negv = jnp.full((16,), NEG_BIG, dtype=jnp.float32)

        def ini(i, _):
            for cch in range(8):
                acc[i, pl.ds(cch * 16, 16)] = negv
            return 0
        lax.fori_loop(0, ROWS_PER + 1, ini, 0)

        def start_block_dma(bi, p):
            pltpu.async_copy(dst_hbm.at[pl.ds(bi * BLK, BLK)],
                             dstblk.at[p], sem_d)
            pltpu.async_copy(src_hbm.at[pl.ds(bi * BLK, BLK)],
                             srcblk.at[p], sem_s)

        def wait_block_dma():
            pltpu.make_async_copy(dst_hbm.at[pl.ds(0, BLK)],
                                  dstblk.at[0], sem_d).wait()
            pltpu.make_async_copy(src_hbm.at[pl.ds(0, BLK)],
                                  srcblk.at[0], sem_s).wait()

        def stage_and_fire(slot0, q):
            for b in range(NBUF):
                @pl.when(q == b)
                def _(b=b):
                    for g2 in range(CHUNK // 16):
                        w16 = cpk[pl.ds(slot0 + g2 * 16, 16)]
                        idxs[b][pl.ds(g2 * 16, 16)] = w16 & 16383
                        dstst[b, pl.ds(g2 * 16, 16)] = w16 >> 14
                    pltpu.async_copy(v_hbm.at[idxs[b]], rows.at[b], sem_g)

        def wait_gather():
            pltpu.make_async_copy(v_hbm.at[idx0],
                                  rows.at[0], sem_g).wait()

        def process(q):
            def grp(g, _):
                rel16 = dstst[q, pl.ds(g * 16, 16)]
                for kk in range(16):
                    r = rel16[kk]
                    rowk = g * 16 + kk
                    avs = [acc[r, pl.ds(c * 16, 16)] for c in range(8)]
                    rvs = [rows[q, rowk, pl.ds(c * 16, 16)] for c in range(8)]
                    for c in range(8):
                        acc[r, pl.ds(c * 16, 16)] = jnp.maximum(avs[c], rvs[c])
                return 0
            lax.fori_loop(0, CHUNK // 16, grp, 0)

        def fire_loop(nch, state):
            def body(ch, st):
                qf, qp, infl = st
                do_proc = infl >= NBUF - 1

                @pl.when(do_proc)
                def _():
                    wait_gather()
                    process(qp & (NBUF - 1))
                stage_and_fire(ch * CHUNK, qf & (NBUF - 1))
                d = do_proc.astype(jnp.int32)
                return (qf + 1, qp + d, infl + 1 - d)
            return lax.fori_loop(0, nch, body, state)

        def block(b, carry):
            cntv, qf, qp, infl = carry
            p = b & 1
            wait_block_dma()

            @pl.when(b + 1 < NBLK)
            def _():
                start_block_dma(b + 1, 1 - p)

            def filt(vi, cntv):
                ms, cums, dstvs, srcvs = [], [], [], []
                for u in range(VEC_UNROLL):
                    off = (vi * VEC_UNROLL + u) * 16
                    dstv = dstblk[p, pl.ds(off, 16)]
                    srcv = srcblk[p, pl.ds(off, 16)]
                    m = (dstv >= lo) & (dstv < hi)
                    cum = plsc.cumsum(m.astype(jnp.int32))
                    ms.append(m)
                    cums.append(cum)
                    dstvs.append(((dstv - lo) << 14) | srcv)
                pref = cntv
                for u in range(VEC_UNROLL):
                    pos = pref + cums[u] - 1
                    plsc.store_scatter(cpk, [pos], dstvs[u], mask=ms[u])
                    pref = pref + _lane_bcast(cums[u], 15)
                return pref
            cntv = lax.fori_loop(0, BLK // 16 // VEC_UNROLL, filt, cntv)
            s = jnp.max(cntv)
            nch = s // CHUNK
            qf, qp, infl = fire_loop(nch, (qf, qp, infl))
            rem = s - nch * CHUNK
            base = nch * CHUNK

            def cp(i, _):
                t1 = cpk[pl.ds(base + i * 16, 16)]
                cpk[pl.ds(i * 16, 16)] = t1
                return 0
            lax.fori_loop(0, (rem + 15) // 16, cp, 0)
            return (lax.broadcast(rem, (16,)), qf, qp, infl)

        start_block_dma(0, 0)
        zero = jnp.int32(0)
        cntv, qf, qp, infl = lax.fori_loop(
            0, NBLK, block, (jnp.zeros((16,), jnp.int32), zero, zero, zero))

        s = jnp.max(cntv)
        dummy_pk = jnp.full((16,), ROWS_PER << 14, dtype=jnp.int32)
        for i in range(CHUNK // 16):
            posf = s + i * 16 + iota16
            plsc.store_scatter(cpk, [posf], dummy_pk)
        qf, qp, infl = fire_loop((s + CHUNK - 1) // CHUNK, (qf, qp, infl))

        def dr(i, qp2):
            wait_gather()
            process(qp2 & (NBUF - 1))
            return qp2 + 1
        lax.fori_loop(0, infl, dr, qp)

        pltpu.sync_copy(acc.at[pl.ds(0, ROWS_PER)],
                        out_hbm.at[pl.ds(lo, ROWS_PER)])

    return k(v, src, dst)


def kernel(x, pos, edge_index, batch, W_msg, b_msg, gn_weight, gn_bias,
           gn_mean_scale, W_fc, b_fc):
    src = edge_index[0]
    dst = edge_index[1]
    W1 = W_msg[:D]
    W2 = W_msg[D:2 * D]
    W3 = W_msg[2 * D:]

    u, v = pl.pallas_call(
        _uv_body,
        out_shape=(jax.ShapeDtypeStruct((N, D), jnp.float32),
                   jax.ShapeDtypeStruct((N, D), jnp.float32)),
    )(x, pos, W1 - W2, -W3, W2, W3)

    segmax = _sc_segmax(v, src, dst)[:N]

    out = pl.pallas_call(
        _final_body,
        out_shape=jax.ShapeDtypeStruct((N, D), jnp.float32),
    )(segmax, u, x, batch.reshape(1, N), b_msg.reshape(1, D),
      gn_weight.reshape(1, D), gn_bias.reshape(1, D),
      gn_mean_scale.reshape(1, D), W_fc, b_fc.reshape(1, D))
    return out


# speedup vs baseline: 1.9726x; 1.0012x over previous
import functools

import jax
import jax.numpy as jnp
from jax import lax
from jax.experimental import pallas as pl
from jax.experimental.pallas import tpu as pltpu
from jax.experimental.pallas import tpu_sc as plsc

N = 10000
E = 320000
D = 128
G = 16
NEG_BIG = -3.0e38

NTILES = 32
ROWS_PER = 320
NPAD = NTILES * ROWS_PER
BLK = 3200
NBLK = E // BLK
VEC_UNROLL = 8
CHUNK = 128
NBUF = 4
CAP = 3600


def _uv_body(x_ref, pos_ref, wxu_ref, wpu_ref, wxv_ref, wpv_ref, u_ref, v_ref):
    x = x_ref[...]
    p = pos_ref[...]
    u_ref[...] = (
        jnp.dot(x, wxu_ref[...], preferred_element_type=jnp.float32)
        + jnp.dot(p, wpu_ref[...], preferred_element_type=jnp.float32)
    )
    v_ref[...] = (
        jnp.dot(x, wxv_ref[...], preferred_element_type=jnp.float32)
        + jnp.dot(p, wpv_ref[...], preferred_element_type=jnp.float32)
    )


def _final_body(segmax_ref, u_ref, x_ref, batch_ref, bmsg_ref, gnw_ref, gnb_ref,
                gnm_ref, wfc_ref, bfc_ref, out_ref):
    segmax = segmax_ref[...]
    agg = jnp.where(segmax <= NEG_BIG,
                    0.0,
                    u_ref[...] + segmax + bmsg_ref[...])
    bt = batch_ref[...]
    gids = jax.lax.broadcasted_iota(jnp.int32, (G, 1), 0)
    onehot = (bt == gids).astype(jnp.float32)
    counts = jnp.maximum(jnp.sum(onehot, axis=1, keepdims=True), 1.0)
    mean = jnp.dot(onehot, agg, preferred_element_type=jnp.float32) / counts
    meanb = jnp.dot(onehot.T, mean * gnm_ref[...], preferred_element_type=jnp.float32)
    out = agg - meanb
    var = jnp.dot(onehot, out * out, preferred_element_type=jnp.float32) / counts
    stdb = jnp.sqrt(jnp.dot(onehot.T, var, preferred_element_type=jnp.float32) + 1e-5)
    out = gnw_ref[...] * out / stdb + gnb_ref[...]
    out = jnp.maximum(out, 0.0)
    out = jnp.dot(out, wfc_ref[...], preferred_element_type=jnp.float32)
    out_ref[...] = out + bfc_ref[...] + x_ref[...]


_GATHER_DNUMS = lax.GatherDimensionNumbers(
    offset_dims=(), collapsed_slice_dims=(0,), start_index_map=(0,))


def _lane_bcast(vec, k):
    idx = jnp.full((16,), k, dtype=jnp.int32)
    return lax.gather(vec, idx[:, None], _GATHER_DNUMS, slice_sizes=(1,),
                      mode=lax.GatherScatterMode.PROMISE_IN_BOUNDS)


def _sc_segmax(v, src, dst):
    mesh = plsc.VectorSubcoreMesh(core_axis_name="c", subcore_axis_name="s")

    @functools.partial(
        pl.kernel,
        out_type=jax.ShapeDtypeStruct((NPAD, D), jnp.float32),
        mesh=mesh,
        compiler_params=pltpu.CompilerParams(needs_layout_passes=False,
                                             use_tc_tiling_on_sc=False),
        scratch_types=[
            pltpu.VMEM((2, BLK), jnp.int32),
            pltpu.VMEM((2, BLK), jnp.int32),
            pltpu.VMEM((CAP,), jnp.int32),
            pltpu.VMEM((CHUNK,), jnp.int32),
            pltpu.VMEM((CHUNK,), jnp.int32),
            pltpu.VMEM((CHUNK,), jnp.int32),
            pltpu.VMEM((CHUNK,), jnp.int32),
            pltpu.VMEM((NBUF, CHUNK, D), jnp.float32),
            pltpu.VMEM((ROWS_PER + 1, D), jnp.float32),
            pltpu.VMEM((NBUF, CHUNK), jnp.int32),
        ] + [
            pltpu.SemaphoreType.DMA,
            pltpu.SemaphoreType.DMA,
            pltpu.SemaphoreType.DMA,
        ],
    )
    def k(v_hbm, src_hbm, dst_hbm, out_hbm,
          dstblk, srcblk, cpk, idx0, idx1, idx2, idx3, rows,
          acc, dstst, sem_d, sem_s, sem_g):
        idxs = (idx0, idx1, idx2, idx3)
        cid = lax.axis_index("c")
        sid = lax.axis_index("s")
        wid = sid * 2 + cid
        iota16 = lax.broadcasted_iota(jnp.int32, (16,), 0)
        lo = wid * ROWS_PER
        hi = lo + ROWS_PER
        negv = jnp.full((16,), NEG_BIG, dtype=jnp.float32)

        def ini(i, _):
            for cch in range(8):
                acc[i, pl.ds(cch * 16, 16)] = negv
            return 0
        lax.fori_loop(0, ROWS_PER + 1, ini, 0)

        def start_block_dma(bi, p):
            pltpu.async_copy(dst_hbm.at[pl.ds(bi * BLK, BLK)],
                             dstblk.at[p], sem_d)
            pltpu.async_copy(src_hbm.at[pl.ds(bi * BLK, BLK)],
                             srcblk.at[p], sem_s)

        def wait_block_dma():
            pltpu.make_async_copy(dst_hbm.at[pl.ds(0, BLK)],
                                  dstblk.at[0], sem_d).wait()
            pltpu.make_async_copy(src_hbm.at[pl.ds(0, BLK)],
                                  srcblk.at[0], sem_s).wait()

        def stage_and_fire(slot0, q):
            for b in range(NBUF):
                @pl.when(q == b)
                def _(b=b):
                    for g2 in range(CHUNK // 16):
                        w16 = cpk[pl.ds(slot0 + g2 * 16, 16)]
                        idxs[b][pl.ds(g2 * 16, 16)] = w16 & 16383
                        dstst[b, pl.ds(g2 * 16, 16)] = w16 >> 14
                    pltpu.async_copy(v_hbm.at[idxs[b]], rows.at[b], sem_g)

        def wait_gather():
            pltpu.make_async_copy(v_hbm.at[idx0],
                                  rows.at[0], sem_g).wait()

        def process(q):
            def grp(g, _):
                rel16 = dstst[q, pl.ds(g * 16, 16)]
                for kk in range(16):
                    r = rel16[kk]
                    rowk = g * 16 + kk
                    avs = [acc[r, pl.ds(c * 16, 16)] for c in range(8)]
                    rvs = [rows[q, rowk, pl.ds(c * 16, 16)] for c in range(8)]
                    for c in range(8):
                        acc[r, pl.ds(c * 16, 16)] = jnp.maximum(avs[c], rvs[c])
                return 0
            lax.fori_loop(0, CHUNK // 16, grp, 0)

        def fire_loop(nch, state):
            def body(ch, st):
                qf, qp, infl = st
                do_proc = infl >= NBUF - 1

                @pl.when(do_proc)
                def _():
                    wait_gather()
                    process(qp & (NBUF - 1))
                stage_and_fire(ch * CHUNK, qf & (NBUF - 1))
                d = do_proc.astype(jnp.int32)
                return (qf + 1, qp + d, infl + 1 - d)
            return lax.fori_loop(0, nch, body, state)

        def block(b, carry):
            cntv, qf, qp, infl = carry
            p = b & 1
            wait_block_dma()

            @pl.when(b + 1 < NBLK)
            def _():
                start_block_dma(b + 1, 1 - p)

            def filt(vi, cntv):
                ms, cums, dstvs, srcvs = [], [], [], []
                for u in range(VEC_UNROLL):
                    off = (vi * VEC_UNROLL + u) * 16
                    dstv = dstblk[p, pl.ds(off, 16)]
                    srcv = srcblk[p, pl.ds(off, 16)]
                    m = (dstv >= lo) & (dstv < hi)
                    cum = plsc.cumsum(m.astype(jnp.int32))
                    ms.append(m)
                    cums.append(cum)
                    dstvs.append(((dstv - lo) << 14) | srcv)
                pref = cntv
                for u in range(VEC_UNROLL):
                    pos = pref + cums[u] - 1
                    plsc.store_scatter(cpk, [pos], dstvs[u], mask=ms[u])
                    pref = pref + _lane_bcast(cums[u], 15)
                return pref
            cntv = lax.fori_loop(0, BLK // 16 // VEC_UNROLL, filt, cntv)
            s = cntv[0]
            nch = s // CHUNK
            qf, qp, infl = fire_loop(nch, (qf, qp, infl))
            rem = s - nch * CHUNK
            base = nch * CHUNK

            def cp(i, _):
                t1 = cpk[pl.ds(base + i * 16, 16)]
                cpk[pl.ds(i * 16, 16)] = t1
                return 0
            lax.fori_loop(0, (rem + 15) // 16, cp, 0)
            return (lax.broadcast(rem, (16,)), qf, qp, infl)

        start_block_dma(0, 0)
        zero = jnp.int32(0)
        cntv, qf, qp, infl = lax.fori_loop(
            0, NBLK, block, (jnp.zeros((16,), jnp.int32), zero, zero, zero))

        s = cntv[0]
        dummy_pk = jnp.full((16,), ROWS_PER << 14, dtype=jnp.int32)
        for i in range(CHUNK // 16):
            posf = s + i * 16 + iota16
            plsc.store_scatter(cpk, [posf], dummy_pk)
        qf, qp, infl = fire_loop((s + CHUNK - 1) // CHUNK, (qf, qp, infl))

        def dr(i, qp2):
            wait_gather()
            process(qp2 & (NBUF - 1))
            return qp2 + 1
        lax.fori_loop(0, infl, dr, qp)

        pltpu.sync_copy(acc.at[pl.ds(0, ROWS_PER)],
                        out_hbm.at[pl.ds(lo, ROWS_PER)])

    return k(v, src, dst)


def kernel(x, pos, edge_index, batch, W_msg, b_msg, gn_weight, gn_bias,
           gn_mean_scale, W_fc, b_fc):
    src = edge_index[0]
    dst = edge_index[1]
    W1 = W_msg[:D]
    W2 = W_msg[D:2 * D]
    W3 = W_msg[2 * D:]

    u, v = pl.pallas_call(
        _uv_body,
        out_shape=(jax.ShapeDtypeStruct((N, D), jnp.float32),
                   jax.ShapeDtypeStruct((N, D), jnp.float32)),
    )(x, pos, W1 - W2, -W3, W2, W3)

    segmax = _sc_segmax(v, src, dst)[:N]

    out = pl.pallas_call(
        _final_body,
        out_shape=jax.ShapeDtypeStruct((N, D), jnp.float32),
    )(segmax, u, x, batch.reshape(1, N), b_msg.reshape(1, D),
      gn_weight.reshape(1, D), gn_bias.reshape(1, D),
      gn_mean_scale.reshape(1, D), W_fc, b_fc.reshape(1, D))
    return out
